# Initial kernel scaffold; baseline (speedup 1.0000x reference)
import jax
import jax.numpy as jnp
from jax.experimental import pallas as pl


def kernel(x, edge_index, W_pre, b_pre, W_s2d, b_s2d, W_d2s, b_d2s, W_lin, b_lin):
    raise NotImplementedError("write your pallas kernel here")


# speedup vs baseline: 9.6437x; 9.6437x over previous
import functools

import jax
import jax.numpy as jnp
from jax import lax
from jax.experimental import pallas as pl
from jax.experimental.pallas import tpu as pltpu
from jax.experimental.pallas import tpu_sc as plsc

NS = 16
B = 80
ROWS = 1000


def _deg_body(eidx_h, z_h, ones_h, out_h, acc, ones_v, iv):
    cid = lax.axis_index("c")
    sid = lax.axis_index("s")
    n = acc.shape[0]
    npt = n // NS
    pltpu.sync_copy(z_h, acc.at[pl.ds(sid * npt, npt)])
    pltpu.sync_copy(ones_h, ones_v)
    plsc.subcore_barrier()
    e = eidx_h.shape[0] // 2
    ept = e // NS
    nb = ept // B

    def step(i, carry):
        off = cid * e + sid * ept + i * B
        pltpu.sync_copy(eidx_h.at[pl.ds(off, B)], iv)
        pltpu.sync_copy(ones_v, acc.at[iv], add=True)
        return carry

    lax.fori_loop(0, nb, step, 0)
    plsc.subcore_barrier()
    pltpu.sync_copy(acc.at[pl.ds(sid * npt, npt)],
                    out_h.at[cid, pl.ds(sid * npt, npt)])


def _agg_body(hsd_h, gidx_h, sidx_h, z_h, out_h, acc, gbuf, gv, sv):
    cid = lax.axis_index("c")
    sid = lax.axis_index("s")
    n = acc.shape[0]
    npt = n // NS
    pltpu.sync_copy(z_h, acc.at[pl.ds(sid * npt, npt)])
    plsc.subcore_barrier()
    e = gidx_h.shape[0] // 2
    ept = e // NS
    nb = ept // B

    def step(i, carry):
        off = cid * e + sid * ept + i * B
        pltpu.sync_copy(gidx_h.at[pl.ds(off, B)], gv)
        pltpu.sync_copy(sidx_h.at[pl.ds(off, B)], sv)
        pltpu.sync_copy(hsd_h.at[gv], gbuf)
        pltpu.sync_copy(gbuf, acc.at[sv], add=True)
        return carry

    lax.fori_loop(0, nb, step, 0)
    plsc.subcore_barrier()
    pltpu.sync_copy(acc.at[pl.ds(sid * npt, npt)],
                    out_h.at[cid, pl.ds(sid * npt, npt)])


def _inv_sqrt(deg):
    return jnp.where(deg > 0.0, lax.rsqrt(deg), 0.0)


def _pre_body(x_ref, w_ref, b_ref, deg_ref, h_ref, hsd_ref):
    h = jnp.maximum(
        jnp.dot(x_ref[...], w_ref[...], preferred_element_type=jnp.float32)
        + b_ref[...], 0.0)
    iinv = _inv_sqrt(deg_ref[1, :, 0:1])
    oinv = _inv_sqrt(deg_ref[0, :, 0:1])
    h_ref[...] = h
    hsd_ref[0] = iinv * h
    hsd_ref[1] = oinv * h


def _post_body(agg_ref, deg_ref, h_ref, ws_ref, bs_ref, wd_ref, bd_ref,
               wl_ref, bl_ref, out_ref):
    oinv = _inv_sqrt(deg_ref[0, :, 0:1])
    iinv = _inv_sqrt(deg_ref[1, :, 0:1])
    aggf = agg_ref[0] * oinv
    aggb = agg_ref[1] * iinv
    conv = (0.5 * (jnp.dot(aggf, ws_ref[...],
                           preferred_element_type=jnp.float32) + bs_ref[...])
            + 0.5 * (jnp.dot(aggb, wd_ref[...],
                             preferred_element_type=jnp.float32) + bd_ref[...]))
    h2 = jnp.maximum(conv + h_ref[...], 0.0)
    out_ref[...] = (jnp.dot(h2, wl_ref[...],
                            preferred_element_type=jnp.float32) + bl_ref[...])


@functools.partial(jax.jit, static_argnums=())
def kernel(x, edge_index, W_pre, b_pre, W_s2d, b_s2d, W_d2s, b_d2s,
           W_lin, b_lin):
    n, d = x.shape
    e = edge_index.shape[1]
    npt = (-(-n // NS) + 7) // 8 * 8
    n_pad = npt * NS
    eidx = edge_index.astype(jnp.int32)
    row = eidx[0]
    col = eidx[1]
    gidx = jnp.concatenate([col, row + n])
    sidx = eidx.reshape(2 * e)
    zeros = jnp.zeros((npt, d), jnp.float32)
    ones = jnp.ones((B, d), jnp.float32)

    mesh = plsc.VectorSubcoreMesh(core_axis_name="c", subcore_axis_name="s")

    deg2 = pl.kernel(
        _deg_body,
        out_type=jax.ShapeDtypeStruct((2, n_pad, d), jnp.float32),
        mesh=mesh,
        scratch_types=[
            pltpu.VMEM_SHARED((n_pad, d), jnp.float32),
            pltpu.VMEM((B, d), jnp.float32),
            pltpu.VMEM((B,), jnp.int32),
        ],
    )(sidx, zeros, ones)

    grid = n // ROWS
    h, hsd = pl.pallas_call(
        _pre_body,
        grid=(grid,),
        in_specs=[
            pl.BlockSpec((ROWS, d), lambda i: (i, 0)),
            pl.BlockSpec((d, d), lambda i: (0, 0)),
            pl.BlockSpec((1, d), lambda i: (0, 0)),
            pl.BlockSpec((2, ROWS, d), lambda i: (0, i, 0)),
        ],
        out_specs=[
            pl.BlockSpec((ROWS, d), lambda i: (i, 0)),
            pl.BlockSpec((2, ROWS, d), lambda i: (0, i, 0)),
        ],
        out_shape=[
            jax.ShapeDtypeStruct((n, d), jnp.float32),
            jax.ShapeDtypeStruct((2, n, d), jnp.float32),
        ],
    )(x, W_pre, b_pre.reshape(1, d), deg2)

    agg2 = pl.kernel(
        _agg_body,
        out_type=jax.ShapeDtypeStruct((2, n_pad, d), jnp.float32),
        mesh=mesh,
        scratch_types=[
            pltpu.VMEM_SHARED((n_pad, d), jnp.float32),
            pltpu.VMEM((B, d), jnp.float32),
            pltpu.VMEM((B,), jnp.int32),
            pltpu.VMEM((B,), jnp.int32),
        ],
    )(hsd.reshape(2 * n, d), gidx, sidx, zeros)

    out = pl.pallas_call(
        _post_body,
        grid=(grid,),
        in_specs=[
            pl.BlockSpec((2, ROWS, d), lambda i: (0, i, 0)),
            pl.BlockSpec((2, ROWS, d), lambda i: (0, i, 0)),
            pl.BlockSpec((ROWS, d), lambda i: (i, 0)),
            pl.BlockSpec((d, d), lambda i: (0, 0)),
            pl.BlockSpec((1, d), lambda i: (0, 0)),
            pl.BlockSpec((d, d), lambda i: (0, 0)),
            pl.BlockSpec((1, d), lambda i: (0, 0)),
            pl.BlockSpec((d, d), lambda i: (0, 0)),
            pl.BlockSpec((1, d), lambda i: (0, 0)),
        ],
        out_specs=pl.BlockSpec((ROWS, d), lambda i: (i, 0)),
        out_shape=jax.ShapeDtypeStruct((n, d), jnp.float32),
    )(agg2, deg2, h, W_s2d, b_s2d.reshape(1, d), W_d2s, b_d2s.reshape(1, d),
      W_lin, b_lin.reshape(1, d))
    return out


# speedup vs baseline: 12.6121x; 1.3078x over previous
import functools

import jax
import jax.numpy as jnp
from jax import lax
from jax.experimental import pallas as pl
from jax.experimental.pallas import tpu as pltpu
from jax.experimental.pallas import tpu_sc as plsc

NS = 16
B = 80
ROWS = 1000


def _deg_body(eidx_h, out_h, hist, idxb, mbuf, res, shist):
    cid = lax.axis_index("c")
    sid = lax.axis_index("s")
    n_pad = shist.shape[1]
    npt = n_pad // NS
    e = eidx_h.shape[0] // 2
    ept = e // NS
    zero16 = jnp.zeros((16,), jnp.float32)
    ones16 = jnp.full((16,), 1.0, jnp.float32)

    def z_step(i, c):
        hist[pl.ds(i * 16, 16)] = zero16
        return c

    lax.fori_loop(0, n_pad // 16, z_step, 0)
    pltpu.sync_copy(eidx_h.at[pl.ds(cid * e + sid * ept, ept)], idxb)

    def h_step(i, c):
        iv = idxb[pl.ds(i * 16, 16)]
        plsc.addupdate_scatter(hist, [iv], ones16)
        return c

    lax.fori_loop(0, ept // 16, h_step, 0)
    pltpu.sync_copy(hist, shist.at[sid])
    plsc.subcore_barrier()

    for k in range(NS):
        pltpu.sync_copy(shist.at[k, pl.ds(sid * npt, npt)], mbuf.at[k])

    def r_step(i, c):
        s = mbuf[0, pl.ds(i * 16, 16)]
        for k in range(1, NS):
            s = s + mbuf[k, pl.ds(i * 16, 16)]
        res[pl.ds(i * 16, 16)] = s
        return c

    lax.fori_loop(0, npt // 16, r_step, 0)
    pltpu.sync_copy(res, out_h.at[cid, pl.ds(sid * npt, npt)])


def _agg_body(hsd_h, gidx_h, sidx_h, z_h, out_h, acc, gbuf, gv, sv):
    cid = lax.axis_index("c")
    sid = lax.axis_index("s")
    n = acc.shape[0]
    npt = n // NS
    pltpu.sync_copy(z_h, acc.at[pl.ds(sid * npt, npt)])
    plsc.subcore_barrier()
    e = gidx_h.shape[0] // 2
    ept = e // NS
    nb = ept // B

    def step(i, carry):
        off = cid * e + sid * ept + i * B
        pltpu.sync_copy(gidx_h.at[pl.ds(off, B)], gv)
        pltpu.sync_copy(sidx_h.at[pl.ds(off, B)], sv)
        pltpu.sync_copy(hsd_h.at[gv], gbuf)
        pltpu.sync_copy(gbuf, acc.at[sv], add=True)
        return carry

    lax.fori_loop(0, nb, step, 0)
    plsc.subcore_barrier()
    pltpu.sync_copy(acc.at[pl.ds(sid * npt, npt)],
                    out_h.at[cid, pl.ds(sid * npt, npt)])


def _inv_sqrt(deg):
    return jnp.where(deg > 0.0, lax.rsqrt(deg), 0.0)


def _pre_body(x_ref, w_ref, b_ref, deg_ref, h_ref, hsd_ref):
    h = jnp.maximum(
        jnp.dot(x_ref[...], w_ref[...], preferred_element_type=jnp.float32)
        + b_ref[...], 0.0)
    iinv = _inv_sqrt(deg_ref[1, :, 0:1])
    oinv = _inv_sqrt(deg_ref[0, :, 0:1])
    h_ref[...] = h
    hsd_ref[0] = iinv * h
    hsd_ref[1] = oinv * h


def _post_body(agg_ref, deg_ref, h_ref, ws_ref, bs_ref, wd_ref, bd_ref,
               wl_ref, bl_ref, out_ref):
    oinv = _inv_sqrt(deg_ref[0, :, 0:1])
    iinv = _inv_sqrt(deg_ref[1, :, 0:1])
    aggf = agg_ref[0] * oinv
    aggb = agg_ref[1] * iinv
    conv = (0.5 * (jnp.dot(aggf, ws_ref[...],
                           preferred_element_type=jnp.float32) + bs_ref[...])
            + 0.5 * (jnp.dot(aggb, wd_ref[...],
                             preferred_element_type=jnp.float32) + bd_ref[...]))
    h2 = jnp.maximum(conv + h_ref[...], 0.0)
    out_ref[...] = (jnp.dot(h2, wl_ref[...],
                            preferred_element_type=jnp.float32) + bl_ref[...])


@functools.partial(jax.jit, static_argnums=())
def kernel(x, edge_index, W_pre, b_pre, W_s2d, b_s2d, W_d2s, b_d2s,
           W_lin, b_lin):
    n, d = x.shape
    e = edge_index.shape[1]
    npt = (-(-n // NS) + 15) // 16 * 16
    n_pad = npt * NS
    ept = e // NS
    eidx = edge_index.astype(jnp.int32)
    row = eidx[0]
    col = eidx[1]
    gidx = jnp.concatenate([col, row + n])
    sidx = eidx.reshape(2 * e)
    zeros = jnp.zeros((npt, d), jnp.float32)

    mesh = plsc.VectorSubcoreMesh(core_axis_name="c", subcore_axis_name="s")

    deg2 = pl.kernel(
        _deg_body,
        out_type=jax.ShapeDtypeStruct((2, n_pad), jnp.float32),
        mesh=mesh,
        compiler_params=pltpu.CompilerParams(needs_layout_passes=False),
        scratch_types=[
            pltpu.VMEM((n_pad,), jnp.float32),
            pltpu.VMEM((ept,), jnp.int32),
            pltpu.VMEM((NS, npt), jnp.float32),
            pltpu.VMEM((npt,), jnp.float32),
            pltpu.VMEM_SHARED((NS, n_pad), jnp.float32),
        ],
    )(sidx)
    deg3 = deg2[:, :n].reshape(2, n, 1)

    grid = n // ROWS
    h, hsd = pl.pallas_call(
        _pre_body,
        grid=(grid,),
        in_specs=[
            pl.BlockSpec((ROWS, d), lambda i: (i, 0)),
            pl.BlockSpec((d, d), lambda i: (0, 0)),
            pl.BlockSpec((1, d), lambda i: (0, 0)),
            pl.BlockSpec((2, ROWS, 1), lambda i: (0, i, 0)),
        ],
        out_specs=[
            pl.BlockSpec((ROWS, d), lambda i: (i, 0)),
            pl.BlockSpec((2, ROWS, d), lambda i: (0, i, 0)),
        ],
        out_shape=[
            jax.ShapeDtypeStruct((n, d), jnp.float32),
            jax.ShapeDtypeStruct((2, n, d), jnp.float32),
        ],
    )(x, W_pre, b_pre.reshape(1, d), deg3)

    agg2 = pl.kernel(
        _agg_body,
        out_type=jax.ShapeDtypeStruct((2, n_pad, d), jnp.float32),
        mesh=mesh,
        scratch_types=[
            pltpu.VMEM_SHARED((n_pad, d), jnp.float32),
            pltpu.VMEM((B, d), jnp.float32),
            pltpu.VMEM((B,), jnp.int32),
            pltpu.VMEM((B,), jnp.int32),
        ],
    )(hsd.reshape(2 * n, d), gidx, sidx, zeros)

    out = pl.pallas_call(
        _post_body,
        grid=(grid,),
        in_specs=[
            pl.BlockSpec((2, ROWS, d), lambda i: (0, i, 0)),
            pl.BlockSpec((2, ROWS, 1), lambda i: (0, i, 0)),
            pl.BlockSpec((ROWS, d), lambda i: (i, 0)),
            pl.BlockSpec((d, d), lambda i: (0, 0)),
            pl.BlockSpec((1, d), lambda i: (0, 0)),
            pl.BlockSpec((d, d), lambda i: (0, 0)),
            pl.BlockSpec((1, d), lambda i: (0, 0)),
            pl.BlockSpec((d, d), lambda i: (0, 0)),
            pl.BlockSpec((1, d), lambda i: (0, 0)),
        ],
        out_specs=pl.BlockSpec((ROWS, d), lambda i: (i, 0)),
        out_shape=jax.ShapeDtypeStruct((n, d), jnp.float32),
    )(agg2, deg3, h, W_s2d, b_s2d.reshape(1, d), W_d2s, b_d2s.reshape(1, d),
      W_lin, b_lin.reshape(1, d))
    return out
